# baseline (speedup 1.0000x reference)
import functools

import jax
import jax.numpy as jnp
from jax import lax
from jax.experimental import pallas as pl
from jax.experimental.pallas import tpu as pltpu
from jax.experimental.pallas import tpu_sc as plsc

N = 10000
E = 320000
FD = 128
NC = 2
NS = 16
NW = NC * NS
EW = E // NW
K = 80
NCH = EW // K
NPAD = 10240
RPS = NPAD // NS

_mesh = plsc.VectorSubcoreMesh(
    core_axis_name="c", subcore_axis_name="s", num_cores=NC, num_subcores=NS)


@functools.partial(
    pl.kernel,
    out_type=jax.ShapeDtypeStruct((NC * NPAD,), jnp.float32),
    mesh=_mesh,
    compiler_params=pltpu.CompilerParams(needs_layout_passes=False),
    scratch_types=[
        pltpu.VMEM((K,), jnp.float32),
        pltpu.VMEM((K,), jnp.float32),
        pltpu.VMEM((K,), jnp.int32),
        pltpu.VMEM((K,), jnp.int32),
        pltpu.VMEM((NPAD,), jnp.float32),
        pltpu.VMEM((NS, RPS), jnp.float32),
        pltpu.VMEM_SHARED((NS, NPAD), jnp.float32),
        pltpu.SemaphoreType.DMA,
        pltpu.SemaphoreType.DMA,
    ],
)
def _sc_deg(w_hbm, col_hbm, out_hbm, w_v0, w_v1, c_v0, c_v1,
            acc1, red, shacc, dsem0, dsem1):
    c = lax.axis_index("c")
    s = lax.axis_index("s")
    wid = c * NS + s
    w_vs = (w_v0, w_v1)
    c_vs = (c_v0, c_v1)
    dsems = (dsem0, dsem1)
    zero16 = jnp.zeros((16,), jnp.float32)

    def start_fetch(i, b):
        base = wid * EW + i * K
        pltpu.async_copy(w_hbm.at[pl.ds(base, K)], w_vs[b], dsems[b])
        pltpu.async_copy(col_hbm.at[pl.ds(base, K)], c_vs[b], dsems[b])

    def wait_fetch(b):
        pltpu.make_async_copy(w_hbm.at[pl.ds(0, K)], w_vs[b], dsems[b]).wait()
        pltpu.make_async_copy(col_hbm.at[pl.ds(0, K)], c_vs[b],
                              dsems[b]).wait()

    start_fetch(0, 0)
    start_fetch(1, 1)

    def z(i, carry):
        acc1[pl.ds(i * 16, 16)] = zero16
        return carry

    lax.fori_loop(0, NPAD // 16, z, 0)

    def half(i, b):
        wait_fetch(b)
        for gi in range(K // 16):
            wv = w_vs[b][pl.ds(gi * 16, 16)]
            cv = c_vs[b][pl.ds(gi * 16, 16)]
            plsc.addupdate_scatter(acc1, [cv], wv)

        @pl.when(i + 2 < NCH)
        def _():
            start_fetch(i + 2, b)

    def pair(t, carry):
        half(2 * t, 0)
        half(2 * t + 1, 1)
        return carry

    lax.fori_loop(0, (NCH - 1) // 2, pair, 0)
    wait_fetch(0)
    for gi in range(K // 16):
        wv = w_v0[pl.ds(gi * 16, 16)]
        cv = c_v0[pl.ds(gi * 16, 16)]
        plsc.addupdate_scatter(acc1, [cv], wv)
    pltpu.sync_copy(acc1, shacc.at[s])
    plsc.subcore_barrier()
    for r in range(NS):
        pltpu.sync_copy(shacc.at[r, pl.ds(s * RPS, RPS)], red.at[r])

    def redloop(b, carry):
        t = red[0, pl.ds(b * 16, 16)]
        for r in range(1, NS):
            t = t + red[r, pl.ds(b * 16, 16)]
        acc1[pl.ds(b * 16, 16)] = t
        return carry

    lax.fori_loop(0, RPS // 16, redloop, 0)
    pltpu.sync_copy(acc1.at[pl.ds(0, RPS)],
                    out_hbm.at[pl.ds(c * NPAD + s * RPS, RPS)])


@functools.partial(
    pl.kernel,
    out_type=jax.ShapeDtypeStruct((NC * NPAD, FD), jnp.float32),
    mesh=_mesh,
    scratch_types=(
        [pltpu.VMEM((K,), jnp.float32) for _ in range(4)]
        + [pltpu.VMEM((K,), jnp.int32) for _ in range(4)]
        + [pltpu.VMEM((K,), jnp.int32) for _ in range(4)]
        + [pltpu.VMEM((K,), jnp.int32) for _ in range(4)]
        + [pltpu.VMEM((K, FD), jnp.float32) for _ in range(4)]
        + [pltpu.VMEM((16,), jnp.float32)]
        + [pltpu.VMEM((16,), jnp.float32)]
        + [pltpu.VMEM_SHARED((NPAD, FD), jnp.float32)]
        + [pltpu.SemaphoreType.DMA for _ in range(12)]
    ),
)
def _sc_msg(g_hbm, row_hbm, col_hbm, w_hbm, out_hbm,
            w_v0, w_v1, w_v2, w_v3, r_v0, r_v1, r_v2, r_v3,
            c_v0, c_v1, c_v2, c_v3, c_s0, c_s1, c_s2, c_s3,
            rows_v0, rows_v1, rows_v2, rows_v3, tbuf, tbuf2, acc,
            isem0, isem1, isem2, isem3, gsem0, gsem1, gsem2, gsem3,
            ssem0, ssem1, ssem2, ssem3):
    c = lax.axis_index("c")
    s = lax.axis_index("s")
    wid = c * NS + s
    w_vs = (w_v0, w_v1, w_v2, w_v3)
    r_vs = (r_v0, r_v1, r_v2, r_v3)
    c_vs = (c_v0, c_v1, c_v2, c_v3)
    c_ss = (c_s0, c_s1, c_s2, c_s3)
    rows_vs = (rows_v0, rows_v1, rows_v2, rows_v3)
    isems = (isem0, isem1, isem2, isem3)
    gsems = (gsem0, gsem1, gsem2, gsem3)
    ssems = (ssem0, ssem1, ssem2, ssem3)
    zero16 = jnp.zeros((16,), jnp.float32)

    def start_idx(i, b):
        base = wid * EW + i * K
        pltpu.async_copy(w_hbm.at[pl.ds(base, K)], w_vs[b], isems[b])
        pltpu.async_copy(row_hbm.at[pl.ds(base, K)], r_vs[b], isems[b])
        pltpu.async_copy(col_hbm.at[pl.ds(base, K)], c_vs[b], isems[b])

    def wait_idx(b):
        pltpu.make_async_copy(w_hbm.at[pl.ds(0, K)], w_vs[b], isems[b]).wait()
        pltpu.make_async_copy(row_hbm.at[pl.ds(0, K)], r_vs[b], isems[b]).wait()
        pltpu.make_async_copy(col_hbm.at[pl.ds(0, K)], c_vs[b], isems[b]).wait()

    def start_gather(b):
        pltpu.async_copy(g_hbm.at[r_vs[b]], rows_vs[b], gsems[b])

    def wait_gather(b):
        pltpu.make_async_copy(g_hbm.at[r_vs[b]], rows_vs[b], gsems[b]).wait()

    def scale(b):
        tbufs = (tbuf, tbuf2)
        for gi in range(K // 16):
            wvec = w_vs[b][pl.ds(gi * 16, 16)]

            def lane(e2, carry, gi=gi, wvec=wvec, b=b):
                for u in range(2):
                    e16 = 2 * e2 + u
                    bvec = wvec.at[jnp.full((16,), e16, jnp.int32)].get(
                        mode="promise_in_bounds")
                    tbufs[u][...] = bvec
                    bv = tbufs[u][...]
                    r = gi * 16 + e16
                    for j in range(FD // 16):
                        rows_vs[b][r, pl.ds(j * 16, 16)] = (
                            rows_vs[b][r, pl.ds(j * 16, 16)] * bv)
                return carry

            lax.fori_loop(0, 8, lane, 0)

    def start_scatter(b):
        for gi in range(K // 16):
            c_ss[b][pl.ds(gi * 16, 16)] = c_vs[b][pl.ds(gi * 16, 16)]
        pltpu.async_copy(rows_vs[b], acc.at[c_ss[b]], ssems[b], add=True)

    def wait_scatter(b):
        pltpu.make_async_copy(rows_vs[b], acc.at[c_ss[b]], ssems[b]).wait()

    for q in range(4):
        start_idx(q, q)
    for r in range(K):
        for j in range(FD // 16):
            rows_v3[r, pl.ds(j * 16, 16)] = zero16
    for j in range(RPS // K):
        pltpu.sync_copy(rows_v3, acc.at[pl.ds(s * RPS + j * K, K)])
    plsc.subcore_barrier()
    wait_idx(0)
    start_gather(0)
    wait_idx(1)
    start_gather(1)

    def half(i, b):
        @pl.when(i >= 2)
        def _():
            wait_scatter((b + 2) % 4)

        @pl.when(i + 2 < NCH)
        def _():
            wait_idx((b + 2) % 4)
            start_gather((b + 2) % 4)
        wait_gather(b)
        scale(b)
        start_scatter(b)

        @pl.when(i + 4 < NCH)
        def _():
            start_idx(i + 4, b)

    def quad(t, carry):
        half(4 * t, 0)
        half(4 * t + 1, 1)
        half(4 * t + 2, 2)
        half(4 * t + 3, 3)
        return carry

    lax.fori_loop(0, (NCH - 1) // 4, quad, 0)
    wait_gather(0)
    wait_scatter(2)
    scale(0)
    start_scatter(0)
    wait_scatter(3)
    wait_scatter(0)

    plsc.subcore_barrier()
    pltpu.sync_copy(acc.at[pl.ds(s * RPS, RPS)],
                    out_hbm.at[pl.ds(c * NPAD + s * RPS, RPS)])


def _tc_prep_body(al_ref, wsc_ref, wfc_ref, x_ref, w0_ref, wmix_ref, h1_ref):
    a = jax.nn.sigmoid(al_ref[0, 0])
    wmix_ref[...] = a * wsc_ref[...] + (1.0 - a) * wfc_ref[...]
    h1_ref[...] = lax.dot_general(
        x_ref[...], w0_ref[...], (((1,), (1,)), ((), ())),
        preferred_element_type=jnp.float32)


def _tc_prep(al, wsc2, wfc2, x, w0):
    return pl.pallas_call(
        _tc_prep_body,
        out_shape=(
            jax.ShapeDtypeStruct((E // FD, FD), jnp.float32),
            jax.ShapeDtypeStruct((N, FD), jnp.float32),
        ),
        in_specs=[
            pl.BlockSpec(memory_space=pltpu.SMEM),
            pl.BlockSpec(memory_space=pltpu.VMEM),
            pl.BlockSpec(memory_space=pltpu.VMEM),
            pl.BlockSpec(memory_space=pltpu.VMEM),
            pl.BlockSpec(memory_space=pltpu.VMEM),
        ],
    )(al, wsc2, wfc2, x, w0)


def _tc_dinv_body(dgp_ref, h1_ref, dinv_ref, g1_ref):
    deg = dgp_ref[0:N, 0:1] + dgp_ref[NPAD:NPAD + N, 0:1] + 1.0
    dinv = lax.rsqrt(deg)
    dinv_ref[...] = dinv
    g1_ref[...] = dinv * h1_ref[...]


def _tc_dinv(dgp, h1):
    return pl.pallas_call(
        _tc_dinv_body,
        out_shape=(
            jax.ShapeDtypeStruct((N, 1), jnp.float32),
            jax.ShapeDtypeStruct((N, FD), jnp.float32),
        ),
    )(dgp, h1)


def _tc_mid_body(p_ref, g_ref, dinv_ref, b_ref, gam_ref, bet_ref, w_ref,
                 out_ref):
    dinv = dinv_ref[...]
    pre = dinv * (p_ref[0:N, :] + p_ref[NPAD:NPAD + N, :] + g_ref[...]) \
        + b_ref[...]
    mu = jnp.mean(pre, axis=0, keepdims=True)
    xc = pre - mu
    var = jnp.mean(xc * xc, axis=0, keepdims=True)
    y = gam_ref[...] * xc * lax.rsqrt(var + 1e-5) + bet_ref[...]
    y = jnp.maximum(y, 0.0)
    h2 = lax.dot_general(y, w_ref[...], (((1,), (1,)), ((), ())),
                         preferred_element_type=jnp.float32)
    out_ref[...] = dinv * h2


def _tc_mid(p, g, dinv, b, gam, bet, w):
    return pl.pallas_call(
        _tc_mid_body,
        out_shape=jax.ShapeDtypeStruct((N, FD), jnp.float32),
    )(p, g, dinv, b, gam, bet, w)


def _tc_final_body(p_ref, g_ref, dinv_ref, b_ref, gam_ref, bet_ref, out_ref):
    dinv = dinv_ref[...]
    pre = dinv * (p_ref[0:N, :] + p_ref[NPAD:NPAD + N, :] + g_ref[...]) \
        + b_ref[...]
    mu = jnp.mean(pre, axis=0, keepdims=True)
    xc = pre - mu
    var = jnp.mean(xc * xc, axis=0, keepdims=True)
    y = gam_ref[...] * xc * lax.rsqrt(var + 1e-5) + bet_ref[...]
    out_ref[...] = jnp.maximum(y, 0.0)


def _tc_final(p, g, dinv, b, gam, bet):
    return pl.pallas_call(
        _tc_final_body,
        out_shape=jax.ShapeDtypeStruct((N, FD), jnp.float32),
    )(p, g, dinv, b, gam, bet)


def kernel(x, edge_index_sc, edge_weight_sc, edge_index_fc, edge_weight_fc,
           alpha, W0, b0, gamma0, beta0, W1, b1, gamma1, beta1):
    row = edge_index_sc[0]
    col = edge_index_sc[1]
    al = alpha.reshape(1, 1)
    wsc2 = edge_weight_sc.reshape(E // FD, FD)
    wfc2 = edge_weight_fc.reshape(E // FD, FD)

    wmix2, h1 = _tc_prep(al, wsc2, wfc2, x, W0)
    wmix = wmix2.reshape(E)

    dgp = _sc_deg(wmix, col).reshape(NC * NPAD, 1)
    dinv, g1 = _tc_dinv(dgp, h1)

    p1 = _sc_msg(g1, row, col, wmix)
    g2 = _tc_mid(p1, g1, dinv, b0.reshape(1, FD), gamma0.reshape(1, FD),
                 beta0.reshape(1, FD), W1)

    p2 = _sc_msg(g2, row, col, wmix)
    return _tc_final(p2, g2, dinv, b1.reshape(1, FD), gamma1.reshape(1, FD),
                     beta1.reshape(1, FD))


# speedup vs baseline: 1.0964x; 1.0964x over previous
import functools

import jax
import jax.numpy as jnp
from jax import lax
from jax.experimental import pallas as pl
from jax.experimental.pallas import tpu as pltpu
from jax.experimental.pallas import tpu_sc as plsc

N = 10000
E = 320000
FD = 128
NC = 2
NS = 16
NW = NC * NS
EW = E // NW
K = 80
NCH = EW // K
NPAD = 10240
RPS = NPAD // NS

_mesh = plsc.VectorSubcoreMesh(
    core_axis_name="c", subcore_axis_name="s", num_cores=NC, num_subcores=NS)


@functools.partial(
    pl.kernel,
    out_type=jax.ShapeDtypeStruct((NC * NPAD,), jnp.float32),
    mesh=_mesh,
    compiler_params=pltpu.CompilerParams(needs_layout_passes=False),
    scratch_types=[
        pltpu.VMEM((K,), jnp.float32),
        pltpu.VMEM((K,), jnp.float32),
        pltpu.VMEM((K,), jnp.int32),
        pltpu.VMEM((K,), jnp.int32),
        pltpu.VMEM((NPAD,), jnp.float32),
        pltpu.VMEM((NS, RPS), jnp.float32),
        pltpu.VMEM_SHARED((NS, NPAD), jnp.float32),
        pltpu.SemaphoreType.DMA,
        pltpu.SemaphoreType.DMA,
    ],
)
def _sc_deg(w_hbm, col_hbm, out_hbm, w_v0, w_v1, c_v0, c_v1,
            acc1, red, shacc, dsem0, dsem1):
    c = lax.axis_index("c")
    s = lax.axis_index("s")
    wid = c * NS + s
    w_vs = (w_v0, w_v1)
    c_vs = (c_v0, c_v1)
    dsems = (dsem0, dsem1)
    zero16 = jnp.zeros((16,), jnp.float32)

    def start_fetch(i, b):
        base = wid * EW + i * K
        pltpu.async_copy(w_hbm.at[pl.ds(base, K)], w_vs[b], dsems[b])
        pltpu.async_copy(col_hbm.at[pl.ds(base, K)], c_vs[b], dsems[b])

    def wait_fetch(b):
        pltpu.make_async_copy(w_hbm.at[pl.ds(0, K)], w_vs[b], dsems[b]).wait()
        pltpu.make_async_copy(col_hbm.at[pl.ds(0, K)], c_vs[b],
                              dsems[b]).wait()

    start_fetch(0, 0)
    start_fetch(1, 1)

    def z(i, carry):
        acc1[pl.ds(i * 16, 16)] = zero16
        return carry

    lax.fori_loop(0, NPAD // 16, z, 0)

    def half(i, b):
        wait_fetch(b)
        for gi in range(K // 16):
            wv = w_vs[b][pl.ds(gi * 16, 16)]
            cv = c_vs[b][pl.ds(gi * 16, 16)]
            plsc.addupdate_scatter(acc1, [cv], wv)

        @pl.when(i + 2 < NCH)
        def _():
            start_fetch(i + 2, b)

    def pair(t, carry):
        half(2 * t, 0)
        half(2 * t + 1, 1)
        return carry

    lax.fori_loop(0, (NCH - 1) // 2, pair, 0)
    wait_fetch(0)
    for gi in range(K // 16):
        wv = w_v0[pl.ds(gi * 16, 16)]
        cv = c_v0[pl.ds(gi * 16, 16)]
        plsc.addupdate_scatter(acc1, [cv], wv)
    pltpu.sync_copy(acc1, shacc.at[s])
    plsc.subcore_barrier()
    for r in range(NS):
        pltpu.sync_copy(shacc.at[r, pl.ds(s * RPS, RPS)], red.at[r])

    def redloop(b, carry):
        t = red[0, pl.ds(b * 16, 16)]
        for r in range(1, NS):
            t = t + red[r, pl.ds(b * 16, 16)]
        acc1[pl.ds(b * 16, 16)] = t
        return carry

    lax.fori_loop(0, RPS // 16, redloop, 0)
    pltpu.sync_copy(acc1.at[pl.ds(0, RPS)],
                    out_hbm.at[pl.ds(c * NPAD + s * RPS, RPS)])


@functools.partial(
    pl.kernel,
    out_type=jax.ShapeDtypeStruct((NC * NPAD, FD), jnp.float32),
    mesh=_mesh,
    scratch_types=(
        [pltpu.VMEM((K,), jnp.float32) for _ in range(4)]
        + [pltpu.VMEM((K,), jnp.int32) for _ in range(4)]
        + [pltpu.VMEM((K,), jnp.int32) for _ in range(4)]
        + [pltpu.VMEM((K,), jnp.int32) for _ in range(4)]
        + [pltpu.VMEM((K, FD), jnp.float32) for _ in range(4)]
        + [pltpu.VMEM((16,), jnp.float32)]
        + [pltpu.VMEM_SHARED((NPAD, FD), jnp.float32)]
        + [pltpu.SemaphoreType.DMA for _ in range(12)]
    ),
)
def _sc_msg(g_hbm, row_hbm, col_hbm, w_hbm, out_hbm,
            w_v0, w_v1, w_v2, w_v3, r_v0, r_v1, r_v2, r_v3,
            c_v0, c_v1, c_v2, c_v3, c_s0, c_s1, c_s2, c_s3,
            rows_v0, rows_v1, rows_v2, rows_v3, tbuf, acc,
            isem0, isem1, isem2, isem3, gsem0, gsem1, gsem2, gsem3,
            ssem0, ssem1, ssem2, ssem3):
    c = lax.axis_index("c")
    s = lax.axis_index("s")
    wid = c * NS + s
    w_vs = (w_v0, w_v1, w_v2, w_v3)
    r_vs = (r_v0, r_v1, r_v2, r_v3)
    c_vs = (c_v0, c_v1, c_v2, c_v3)
    c_ss = (c_s0, c_s1, c_s2, c_s3)
    rows_vs = (rows_v0, rows_v1, rows_v2, rows_v3)
    isems = (isem0, isem1, isem2, isem3)
    gsems = (gsem0, gsem1, gsem2, gsem3)
    ssems = (ssem0, ssem1, ssem2, ssem3)
    zero16 = jnp.zeros((16,), jnp.float32)

    def start_idx(i, b):
        base = wid * EW + i * K
        pltpu.async_copy(w_hbm.at[pl.ds(base, K)], w_vs[b], isems[b])
        pltpu.async_copy(row_hbm.at[pl.ds(base, K)], r_vs[b], isems[b])
        pltpu.async_copy(col_hbm.at[pl.ds(base, K)], c_vs[b], isems[b])

    def wait_idx(b):
        pltpu.make_async_copy(w_hbm.at[pl.ds(0, K)], w_vs[b], isems[b]).wait()
        pltpu.make_async_copy(row_hbm.at[pl.ds(0, K)], r_vs[b], isems[b]).wait()
        pltpu.make_async_copy(col_hbm.at[pl.ds(0, K)], c_vs[b], isems[b]).wait()

    def start_gather(b):
        pltpu.async_copy(g_hbm.at[r_vs[b]], rows_vs[b], gsems[b])

    def wait_gather(b):
        pltpu.make_async_copy(g_hbm.at[r_vs[b]], rows_vs[b], gsems[b]).wait()

    def scale(b):
        for gi in range(K // 16):
            wvec = w_vs[b][pl.ds(gi * 16, 16)]

            def lane(e16, carry, gi=gi, wvec=wvec, b=b):
                bv = wvec.at[jnp.full((16,), e16, jnp.int32)].get(
                    mode="promise_in_bounds")
                r = gi * 16 + e16
                for j in range(FD // 16):
                    rows_vs[b][r, pl.ds(j * 16, 16)] = (
                        rows_vs[b][r, pl.ds(j * 16, 16)] * bv)
                return carry

            lax.fori_loop(0, 16, lane, 0)

    def start_scatter(b):
        for gi in range(K // 16):
            c_ss[b][pl.ds(gi * 16, 16)] = c_vs[b][pl.ds(gi * 16, 16)]
        pltpu.async_copy(rows_vs[b], acc.at[c_ss[b]], ssems[b], add=True)

    def wait_scatter(b):
        pltpu.make_async_copy(rows_vs[b], acc.at[c_ss[b]], ssems[b]).wait()

    for q in range(4):
        start_idx(q, q)
    for r in range(K):
        for j in range(FD // 16):
            rows_v3[r, pl.ds(j * 16, 16)] = zero16
    for j in range(RPS // K):
        pltpu.sync_copy(rows_v3, acc.at[pl.ds(s * RPS + j * K, K)])
    plsc.subcore_barrier()
    wait_idx(0)
    start_gather(0)
    wait_idx(1)
    start_gather(1)

    def half(i, b):
        @pl.when(i >= 2)
        def _():
            wait_scatter((b + 2) % 4)

        @pl.when(i + 2 < NCH)
        def _():
            wait_idx((b + 2) % 4)
            start_gather((b + 2) % 4)
        wait_gather(b)
        scale(b)
        start_scatter(b)

        @pl.when(i + 4 < NCH)
        def _():
            start_idx(i + 4, b)

    def quad(t, carry):
        half(4 * t, 0)
        half(4 * t + 1, 1)
        half(4 * t + 2, 2)
        half(4 * t + 3, 3)
        return carry

    lax.fori_loop(0, (NCH - 1) // 4, quad, 0)
    wait_gather(0)
    wait_scatter(2)
    scale(0)
    start_scatter(0)
    wait_scatter(3)
    wait_scatter(0)

    plsc.subcore_barrier()
    pltpu.sync_copy(acc.at[pl.ds(s * RPS, RPS)],
                    out_hbm.at[pl.ds(c * NPAD + s * RPS, RPS)])


def _tc_prep_body(al_ref, wsc_ref, wfc_ref, x_ref, w0_ref, wmix_ref, h1_ref):
    a = jax.nn.sigmoid(al_ref[0, 0])
    wmix_ref[...] = a * wsc_ref[...] + (1.0 - a) * wfc_ref[...]
    h1_ref[...] = lax.dot_general(
        x_ref[...], w0_ref[...], (((1,), (1,)), ((), ())),
        preferred_element_type=jnp.float32)


def _tc_prep(al, wsc2, wfc2, x, w0):
    return pl.pallas_call(
        _tc_prep_body,
        out_shape=(
            jax.ShapeDtypeStruct((E // FD, FD), jnp.float32),
            jax.ShapeDtypeStruct((N, FD), jnp.float32),
        ),
        in_specs=[
            pl.BlockSpec(memory_space=pltpu.SMEM),
            pl.BlockSpec(memory_space=pltpu.VMEM),
            pl.BlockSpec(memory_space=pltpu.VMEM),
            pl.BlockSpec(memory_space=pltpu.VMEM),
            pl.BlockSpec(memory_space=pltpu.VMEM),
        ],
    )(al, wsc2, wfc2, x, w0)


def _tc_dinv_body(dgp_ref, h1_ref, dinv_ref, g1_ref):
    deg = dgp_ref[0:N, 0:1] + dgp_ref[NPAD:NPAD + N, 0:1] + 1.0
    dinv = lax.rsqrt(deg)
    dinv_ref[...] = dinv
    g1_ref[...] = dinv * h1_ref[...]


def _tc_dinv(dgp, h1):
    return pl.pallas_call(
        _tc_dinv_body,
        out_shape=(
            jax.ShapeDtypeStruct((N, 1), jnp.float32),
            jax.ShapeDtypeStruct((N, FD), jnp.float32),
        ),
    )(dgp, h1)


def _tc_mid_body(p_ref, g_ref, dinv_ref, b_ref, gam_ref, bet_ref, w_ref,
                 out_ref):
    dinv = dinv_ref[...]
    pre = dinv * (p_ref[0:N, :] + p_ref[NPAD:NPAD + N, :] + g_ref[...]) \
        + b_ref[...]
    mu = jnp.mean(pre, axis=0, keepdims=True)
    xc = pre - mu
    var = jnp.mean(xc * xc, axis=0, keepdims=True)
    y = gam_ref[...] * xc * lax.rsqrt(var + 1e-5) + bet_ref[...]
    y = jnp.maximum(y, 0.0)
    h2 = lax.dot_general(y, w_ref[...], (((1,), (1,)), ((), ())),
                         preferred_element_type=jnp.float32)
    out_ref[...] = dinv * h2


def _tc_mid(p, g, dinv, b, gam, bet, w):
    return pl.pallas_call(
        _tc_mid_body,
        out_shape=jax.ShapeDtypeStruct((N, FD), jnp.float32),
    )(p, g, dinv, b, gam, bet, w)


def _tc_final_body(p_ref, g_ref, dinv_ref, b_ref, gam_ref, bet_ref, out_ref):
    dinv = dinv_ref[...]
    pre = dinv * (p_ref[0:N, :] + p_ref[NPAD:NPAD + N, :] + g_ref[...]) \
        + b_ref[...]
    mu = jnp.mean(pre, axis=0, keepdims=True)
    xc = pre - mu
    var = jnp.mean(xc * xc, axis=0, keepdims=True)
    y = gam_ref[...] * xc * lax.rsqrt(var + 1e-5) + bet_ref[...]
    out_ref[...] = jnp.maximum(y, 0.0)


def _tc_final(p, g, dinv, b, gam, bet):
    return pl.pallas_call(
        _tc_final_body,
        out_shape=jax.ShapeDtypeStruct((N, FD), jnp.float32),
    )(p, g, dinv, b, gam, bet)


def kernel(x, edge_index_sc, edge_weight_sc, edge_index_fc, edge_weight_fc,
           alpha, W0, b0, gamma0, beta0, W1, b1, gamma1, beta1):
    row = edge_index_sc[0]
    col = edge_index_sc[1]
    al = alpha.reshape(1, 1)
    wsc2 = edge_weight_sc.reshape(E // FD, FD)
    wfc2 = edge_weight_fc.reshape(E // FD, FD)

    wmix2, h1 = _tc_prep(al, wsc2, wfc2, x, W0)
    wmix = wmix2.reshape(E)

    dgp = _sc_deg(wmix, col).reshape(NC * NPAD, 1)
    dinv, g1 = _tc_dinv(dgp, h1)

    p1 = _sc_msg(g1, row, col, wmix)
    g2 = _tc_mid(p1, g1, dinv, b0.reshape(1, FD), gamma0.reshape(1, FD),
                 beta0.reshape(1, FD), W1)

    p2 = _sc_msg(g2, row, col, wmix)
    return _tc_final(p2, g2, dinv, b1.reshape(1, FD), gamma1.reshape(1, FD),
                     beta1.reshape(1, FD))
